# baseline (speedup 1.0000x reference)
import functools

import jax
import jax.numpy as jnp
from jax import lax
from jax.experimental import pallas as pl
from jax.experimental.pallas import tpu as pltpu
from jax.experimental.pallas import tpu_sc as plsc

NC = 2
NS = 16
NW = NC * NS
BATCH = 128
IG = 4
KS = 4


def _cnt_kernel(n_pad, d, b_tile):
    mesh = plsc.VectorSubcoreMesh(core_axis_name="c", subcore_axis_name="s")
    rpt = n_pad // NS

    @functools.partial(
        pl.kernel, mesh=mesh,
        out_type=[jax.ShapeDtypeStruct((NC, n_pad, d), jnp.float32)],
        scratch_types=[
            pltpu.VMEM((b_tile, BATCH), jnp.int32),
            pltpu.VMEM((BATCH, d), jnp.float32),
            pltpu.VMEM_SHARED((n_pad, d), jnp.float32),
        ] + [pltpu.SemaphoreType.DMA for _ in range(KS)])
    def cntk(dsts, ones_hbm, zrow, cnt_o, dst_a, ones_v, cnt_sh, *sems):
        c = lax.axis_index("c")
        s = lax.axis_index("s")
        wid = s * NC + c
        pltpu.sync_copy(dsts.at[wid], dst_a)
        pltpu.sync_copy(ones_hbm, ones_v)
        pltpu.sync_copy(zrow, cnt_sh.at[pl.ds(s * rpt, rpt)])
        plsc.subcore_barrier()

        def body(q, carry):
            for k in range(KS):
                j = q * KS + k

                @pl.when(q > 0)
                def _():
                    pltpu.make_async_copy(
                        ones_v, cnt_sh.at[dst_a.at[j]], sems[k]).wait()

                pltpu.async_copy(ones_v, cnt_sh.at[dst_a.at[j]],
                                 sems[k], add=True)
            return carry

        lax.fori_loop(0, b_tile // KS, body, 0)
        for k in range(KS):
            pltpu.make_async_copy(
                ones_v, cnt_sh.at[dst_a.at[b_tile - KS + k]],
                sems[k]).wait()
        plsc.subcore_barrier()
        pltpu.sync_copy(cnt_sh.at[pl.ds(s * rpt, rpt)],
                        cnt_o.at[c, pl.ds(s * rpt, rpt)])
    return cntk


def _seg_sum_kernel(n_pad, d, b_tile, with_cnt=False):
    mesh = plsc.VectorSubcoreMesh(core_axis_name="c", subcore_axis_name="s")
    rpt = n_pad // NS

    assert b_tile % IG == 0
    nq = b_tile // IG
    out_type = [jax.ShapeDtypeStruct((NC, n_pad, d), jnp.float32)]
    scratch = [
        pltpu.VMEM((2 * IG, BATCH), jnp.int32),
        pltpu.VMEM((BATCH, d), jnp.float32),
        pltpu.VMEM((BATCH, d), jnp.float32),
        pltpu.VMEM_SHARED((n_pad, d), jnp.float32),
        pltpu.SemaphoreType.DMA,
        pltpu.SemaphoreType.DMA,
        pltpu.SemaphoreType.DMA,
        pltpu.SemaphoreType.DMA,
    ]

    @functools.partial(pl.kernel, mesh=mesh, out_type=out_type,
                       scratch_types=scratch)
    def seg(tab, sd, zrow, part_o, sd_v, rows_a, rows_b, acc_sh,
            gsem_a, gsem_b, ssem_a, ssem_b):
        c = lax.axis_index("c")
        s = lax.axis_index("s")
        wid = s * NC + c
        rows = (rows_a, rows_b)
        gsem = (gsem_a, gsem_b)
        ssem = (ssem_a, ssem_b)

        pltpu.sync_copy(zrow, acc_sh.at[pl.ds(s * rpt, rpt)])
        plsc.subcore_barrier()

        def issue_g(b, p):
            pltpu.async_copy(tab.at[sd_v.at[b]], rows[p], gsem[p])

        def wait_g(b, p):
            pltpu.make_async_copy(tab.at[sd_v.at[b]], rows[p],
                                  gsem[p]).wait()

        def issue_s(b, p):
            pltpu.async_copy(rows[p], acc_sh.at[sd_v.at[IG + b]],
                             ssem[p], add=True)

        def wait_s(b, p):
            pltpu.make_async_copy(rows[p], acc_sh.at[sd_v.at[IG + b]],
                                  ssem[p]).wait()

        def body(g, carry):
            pltpu.sync_copy(sd.at[wid, g], sd_v)
            issue_g(0, 0)
            issue_g(1, 1)
            for b in range(IG):
                p = b % 2
                wait_g(b, p)
                issue_s(b, p)
                if 1 <= b and b + 1 < IG:
                    wait_s(b - 1, 1 - p)
                    issue_g(b + 1, 1 - p)
            wait_s(IG - 2, IG % 2)
            wait_s(IG - 1, 1 - IG % 2)
            return carry

        lax.fori_loop(0, nq, body, 0)
        plsc.subcore_barrier()
        pltpu.sync_copy(acc_sh.at[pl.ds(s * rpt, rpt)],
                        part_o.at[c, pl.ds(s * rpt, rpt)])
    return seg


def _prep_body(x_ref, w_ref, b_ref, a_ref, r_ref):
    y = jnp.dot(x_ref[...], w_ref[...],
                preferred_element_type=jnp.float32,
                precision=lax.Precision.HIGHEST)
    d = a_ref.shape[-1]
    a_ref[...] = y[:, :d]
    r_ref[...] = y[:, d:] + b_ref[...]


def _mid_body(p_ref, pc_ref, r_ref, w_ref, b_ref, a_ref, r2_ref):
    cnt = pc_ref[0, :, 0:1] + pc_ref[1, :, 0:1]
    inv = 1.0 / jnp.maximum(cnt, 1.0)
    h = jnp.maximum((p_ref[0] + p_ref[1]) * inv + r_ref[...], 0.0)
    y = jnp.dot(h, w_ref[...],
                preferred_element_type=jnp.float32,
                precision=lax.Precision.HIGHEST)
    d = a_ref.shape[-1]
    a_ref[...] = y[:, :d]
    r2_ref[...] = y[:, d:] + b_ref[...]


def _post_body(p_ref, pc_ref, r_ref, o_ref):
    cnt = pc_ref[0, :, 0:1] + pc_ref[1, :, 0:1]
    inv = 1.0 / jnp.maximum(cnt, 1.0)
    o_ref[...] = (p_ref[0] + p_ref[1]) * inv + r_ref[...]


def kernel(x, edge_index, edge_weight, Wl1, bl1, Wr1,
           Wl2, bl2, Wr2, Wl3, bl3, Wr3):
    n, d_in = x.shape
    e = edge_index.shape[1]
    d_hid = Wl1.shape[0]
    d_out = Wl3.shape[0]
    n_pad = -(-(n + 1) // 128) * 128
    ndump = n_pad - n

    src = edge_index[0].astype(jnp.int32)
    dst = edge_index[1].astype(jnp.int32)

    unit = NW * BATCH
    b_tile = -(-e // unit)
    b_tile = -(-b_tile // 8) * 8
    e_pad = NW * b_tile * BATCH
    pad = e_pad - e
    pad_ar = jnp.arange(pad, dtype=jnp.int32)
    src_p = jnp.concatenate([src, pad_ar % n])
    dst_p = jnp.concatenate([dst, n + (pad_ar % ndump)])
    srcs = src_p.reshape(NW, b_tile, BATCH)
    dsts = dst_p.reshape(NW, b_tile, BATCH)
    nq = b_tile // IG
    sd = jnp.concatenate([srcs.reshape(NW, nq, IG, BATCH),
                          dsts.reshape(NW, nq, IG, BATCH)], axis=2)

    rpt = n_pad // NS
    zrow = jnp.zeros((rpt, d_hid), jnp.float32)

    cntk = _cnt_kernel(n_pad, d_hid, b_tile)
    seg = _seg_sum_kernel(n_pad, d_hid, b_tile)
    ones128 = jnp.ones((BATCH, d_hid), jnp.float32)

    bn = 1000
    grid = (n // bn,)

    def prep_call(h, Wl, Wr, bl, dh):
        wcat = jnp.concatenate([Wl.T, Wr.T], axis=1)
        return pl.pallas_call(
            _prep_body,
            grid=grid,
            in_specs=[
                pl.BlockSpec((bn, h.shape[1]), lambda i: (i, 0)),
                pl.BlockSpec((h.shape[1], 2 * dh), lambda i: (0, 0)),
                pl.BlockSpec((1, dh), lambda i: (0, 0)),
            ],
            out_specs=[
                pl.BlockSpec((bn, dh), lambda i: (i, 0)),
                pl.BlockSpec((bn, dh), lambda i: (i, 0)),
            ],
            out_shape=[jax.ShapeDtypeStruct((n, dh), jnp.float32)] * 2,
        )(h, wcat, bl.reshape(1, dh))

    def mid_call(part, cntp, r, Wl, Wr, bl, dh):
        wcat = jnp.concatenate([Wl.T, Wr.T], axis=1)
        d_prev = part.shape[2]
        return pl.pallas_call(
            _mid_body,
            grid=grid,
            in_specs=[
                pl.BlockSpec((NC, bn, d_prev), lambda i: (0, i, 0)),
                pl.BlockSpec((NC, bn, 128), lambda i: (0, i, 0)),
                pl.BlockSpec((bn, d_prev), lambda i: (i, 0)),
                pl.BlockSpec((d_prev, 2 * dh), lambda i: (0, 0)),
                pl.BlockSpec((1, dh), lambda i: (0, 0)),
            ],
            out_specs=[
                pl.BlockSpec((bn, dh), lambda i: (i, 0)),
                pl.BlockSpec((bn, dh), lambda i: (i, 0)),
            ],
            out_shape=[jax.ShapeDtypeStruct((n, dh), jnp.float32)] * 2,
        )(part, cntp, r, wcat, bl.reshape(1, dh))

    def post_call(part, cntp, r):
        d_prev = part.shape[2]
        return pl.pallas_call(
            _post_body,
            grid=grid,
            in_specs=[
                pl.BlockSpec((NC, bn, d_prev), lambda i: (0, i, 0)),
                pl.BlockSpec((NC, bn, 128), lambda i: (0, i, 0)),
                pl.BlockSpec((bn, d_prev), lambda i: (i, 0)),
            ],
            out_specs=pl.BlockSpec((bn, d_prev), lambda i: (i, 0)),
            out_shape=jax.ShapeDtypeStruct((n, d_prev), jnp.float32),
        )(part, cntp, r)

    def seg1(*args):
        res = seg(*args)
        return res[0] if isinstance(res, (list, tuple)) else res

    cntp = cntk(dsts, ones128, zrow)
    cntp = cntp[0] if isinstance(cntp, (list, tuple)) else cntp
    a1, r1 = prep_call(x, Wl1, Wr1, bl1, d_hid)
    part1 = seg1(a1, sd, zrow)

    a2, r2 = mid_call(part1, cntp, r1, Wl2, Wr2, bl2, d_hid)
    part2 = seg1(a2, sd, zrow)
    a3, r3 = mid_call(part2, cntp, r2, Wl3, Wr3, bl3, d_out)
    part3 = seg1(a3, sd, zrow)
    out = post_call(part3, cntp, r3)
    return out


# speedup vs baseline: 1.2023x; 1.2023x over previous
import functools

import jax
import jax.numpy as jnp
from jax import lax
from jax.experimental import pallas as pl
from jax.experimental.pallas import tpu as pltpu
from jax.experimental.pallas import tpu_sc as plsc

NC = 2
NS = 16
NW = NC * NS
BATCH = 128
IG = 8
KS = 4


def _cnt_kernel(n_pad, d, b_tile):
    mesh = plsc.VectorSubcoreMesh(core_axis_name="c", subcore_axis_name="s")
    rpt = n_pad // NS

    @functools.partial(
        pl.kernel, mesh=mesh,
        out_type=[jax.ShapeDtypeStruct((NC, n_pad, d), jnp.float32)],
        scratch_types=[
            pltpu.VMEM((b_tile, BATCH), jnp.int32),
            pltpu.VMEM((BATCH, d), jnp.float32),
            pltpu.VMEM_SHARED((n_pad, d), jnp.float32),
        ] + [pltpu.SemaphoreType.DMA for _ in range(KS)])
    def cntk(dsts, ones_hbm, zrow, cnt_o, dst_a, ones_v, cnt_sh, *sems):
        c = lax.axis_index("c")
        s = lax.axis_index("s")
        wid = s * NC + c
        pltpu.sync_copy(dsts.at[wid], dst_a)
        pltpu.sync_copy(ones_hbm, ones_v)
        pltpu.sync_copy(zrow, cnt_sh.at[pl.ds(s * rpt, rpt)])
        plsc.subcore_barrier()

        def body(q, carry):
            for k in range(KS):
                j = q * KS + k

                @pl.when(q > 0)
                def _():
                    pltpu.make_async_copy(
                        ones_v, cnt_sh.at[dst_a.at[j]], sems[k]).wait()

                pltpu.async_copy(ones_v, cnt_sh.at[dst_a.at[j]],
                                 sems[k], add=True)
            return carry

        lax.fori_loop(0, b_tile // KS, body, 0)
        for k in range(KS):
            pltpu.make_async_copy(
                ones_v, cnt_sh.at[dst_a.at[b_tile - KS + k]],
                sems[k]).wait()
        plsc.subcore_barrier()
        pltpu.sync_copy(cnt_sh.at[pl.ds(s * rpt, rpt)],
                        cnt_o.at[c, pl.ds(s * rpt, rpt)])
    return cntk


def _seg_sum_kernel(n_pad, d, b_tile, with_cnt=False):
    mesh = plsc.VectorSubcoreMesh(core_axis_name="c", subcore_axis_name="s")
    rpt = n_pad // NS

    assert b_tile % IG == 0
    nq = b_tile // IG
    out_type = [jax.ShapeDtypeStruct((NC, n_pad, d), jnp.float32)]
    scratch = [
        pltpu.VMEM((2 * IG, BATCH), jnp.int32),
        pltpu.VMEM((BATCH, d), jnp.float32),
        pltpu.VMEM((BATCH, d), jnp.float32),
        pltpu.VMEM_SHARED((n_pad, d), jnp.float32),
        pltpu.SemaphoreType.DMA,
        pltpu.SemaphoreType.DMA,
    ]

    @functools.partial(pl.kernel, mesh=mesh, out_type=out_type,
                       scratch_types=scratch)
    def seg(tab, sd, zrow, part_o, sd_v, rows_a, rows_b, acc_sh,
            sem_a, sem_b):
        c = lax.axis_index("c")
        s = lax.axis_index("s")
        wid = s * NC + c
        rows = (rows_a, rows_b)
        sems = (sem_a, sem_b)

        pltpu.sync_copy(zrow, acc_sh.at[pl.ds(s * rpt, rpt)])
        plsc.subcore_barrier()

        def body(g, carry):
            pltpu.sync_copy(sd.at[wid, g], sd_v)
            pltpu.async_copy(tab.at[sd_v.at[0]], rows[0], sems[0])
            for b in range(IG):
                p = b % 2
                if b + 1 < IG:
                    pltpu.async_copy(tab.at[sd_v.at[b + 1]],
                                     rows[1 - p], sems[1 - p])
                pltpu.make_async_copy(tab.at[sd_v.at[b]],
                                      rows[p], sems[p]).wait()
                pltpu.sync_copy(rows[p], acc_sh.at[sd_v.at[IG + b]],
                                add=True)
            return carry

        lax.fori_loop(0, nq, body, 0)
        plsc.subcore_barrier()
        pltpu.sync_copy(acc_sh.at[pl.ds(s * rpt, rpt)],
                        part_o.at[c, pl.ds(s * rpt, rpt)])
    return seg


def _prep_body(x_ref, w_ref, b_ref, a_ref, r_ref):
    y = jnp.dot(x_ref[...], w_ref[...],
                preferred_element_type=jnp.float32,
                precision=lax.Precision.HIGHEST)
    d = a_ref.shape[-1]
    a_ref[...] = y[:, :d]
    r_ref[...] = y[:, d:] + b_ref[...]


def _mid_body(p_ref, pc_ref, r_ref, w_ref, b_ref, a_ref, r2_ref):
    cnt = pc_ref[0, :, 0:1] + pc_ref[1, :, 0:1]
    inv = 1.0 / jnp.maximum(cnt, 1.0)
    h = jnp.maximum((p_ref[0] + p_ref[1]) * inv + r_ref[...], 0.0)
    y = jnp.dot(h, w_ref[...],
                preferred_element_type=jnp.float32,
                precision=lax.Precision.HIGHEST)
    d = a_ref.shape[-1]
    a_ref[...] = y[:, :d]
    r2_ref[...] = y[:, d:] + b_ref[...]


def _post_body(p_ref, pc_ref, r_ref, o_ref):
    cnt = pc_ref[0, :, 0:1] + pc_ref[1, :, 0:1]
    inv = 1.0 / jnp.maximum(cnt, 1.0)
    o_ref[...] = (p_ref[0] + p_ref[1]) * inv + r_ref[...]


def kernel(x, edge_index, edge_weight, Wl1, bl1, Wr1,
           Wl2, bl2, Wr2, Wl3, bl3, Wr3):
    n, d_in = x.shape
    e = edge_index.shape[1]
    d_hid = Wl1.shape[0]
    d_out = Wl3.shape[0]
    n_pad = -(-(n + 1) // 128) * 128
    ndump = n_pad - n

    src = edge_index[0].astype(jnp.int32)
    dst = edge_index[1].astype(jnp.int32)

    unit = NW * BATCH
    b_tile = -(-e // unit)
    b_tile = -(-b_tile // 8) * 8
    e_pad = NW * b_tile * BATCH
    pad = e_pad - e
    pad_ar = jnp.arange(pad, dtype=jnp.int32)
    src_p = jnp.concatenate([src, pad_ar % n])
    dst_p = jnp.concatenate([dst, n + (pad_ar % ndump)])
    srcs = src_p.reshape(NW, b_tile, BATCH)
    dsts = dst_p.reshape(NW, b_tile, BATCH)
    nq = b_tile // IG
    sd = jnp.concatenate([srcs.reshape(NW, nq, IG, BATCH),
                          dsts.reshape(NW, nq, IG, BATCH)], axis=2)

    rpt = n_pad // NS
    zrow = jnp.zeros((rpt, d_hid), jnp.float32)

    cntk = _cnt_kernel(n_pad, d_hid, b_tile)
    seg = _seg_sum_kernel(n_pad, d_hid, b_tile)
    ones128 = jnp.ones((BATCH, d_hid), jnp.float32)

    bn = 1000
    grid = (n // bn,)

    def prep_call(h, Wl, Wr, bl, dh):
        wcat = jnp.concatenate([Wl.T, Wr.T], axis=1)
        return pl.pallas_call(
            _prep_body,
            grid=grid,
            in_specs=[
                pl.BlockSpec((bn, h.shape[1]), lambda i: (i, 0)),
                pl.BlockSpec((h.shape[1], 2 * dh), lambda i: (0, 0)),
                pl.BlockSpec((1, dh), lambda i: (0, 0)),
            ],
            out_specs=[
                pl.BlockSpec((bn, dh), lambda i: (i, 0)),
                pl.BlockSpec((bn, dh), lambda i: (i, 0)),
            ],
            out_shape=[jax.ShapeDtypeStruct((n, dh), jnp.float32)] * 2,
        )(h, wcat, bl.reshape(1, dh))

    def mid_call(part, cntp, r, Wl, Wr, bl, dh):
        wcat = jnp.concatenate([Wl.T, Wr.T], axis=1)
        d_prev = part.shape[2]
        return pl.pallas_call(
            _mid_body,
            grid=grid,
            in_specs=[
                pl.BlockSpec((NC, bn, d_prev), lambda i: (0, i, 0)),
                pl.BlockSpec((NC, bn, 128), lambda i: (0, i, 0)),
                pl.BlockSpec((bn, d_prev), lambda i: (i, 0)),
                pl.BlockSpec((d_prev, 2 * dh), lambda i: (0, 0)),
                pl.BlockSpec((1, dh), lambda i: (0, 0)),
            ],
            out_specs=[
                pl.BlockSpec((bn, dh), lambda i: (i, 0)),
                pl.BlockSpec((bn, dh), lambda i: (i, 0)),
            ],
            out_shape=[jax.ShapeDtypeStruct((n, dh), jnp.float32)] * 2,
        )(part, cntp, r, wcat, bl.reshape(1, dh))

    def post_call(part, cntp, r):
        d_prev = part.shape[2]
        return pl.pallas_call(
            _post_body,
            grid=grid,
            in_specs=[
                pl.BlockSpec((NC, bn, d_prev), lambda i: (0, i, 0)),
                pl.BlockSpec((NC, bn, 128), lambda i: (0, i, 0)),
                pl.BlockSpec((bn, d_prev), lambda i: (i, 0)),
            ],
            out_specs=pl.BlockSpec((bn, d_prev), lambda i: (i, 0)),
            out_shape=jax.ShapeDtypeStruct((n, d_prev), jnp.float32),
        )(part, cntp, r)

    def seg1(*args):
        res = seg(*args)
        return res[0] if isinstance(res, (list, tuple)) else res

    cntp = cntk(dsts, ones128, zrow)
    cntp = cntp[0] if isinstance(cntp, (list, tuple)) else cntp
    a1, r1 = prep_call(x, Wl1, Wr1, bl1, d_hid)
    part1 = seg1(a1, sd, zrow)

    a2, r2 = mid_call(part1, cntp, r1, Wl2, Wr2, bl2, d_hid)
    part2 = seg1(a2, sd, zrow)
    a3, r3 = mid_call(part2, cntp, r2, Wl3, Wr3, bl3, d_out)
    part3 = seg1(a3, sd, zrow)
    out = post_call(part3, cntp, r3)
    return out


# speedup vs baseline: 1.2674x; 1.0542x over previous
import functools

import jax
import jax.numpy as jnp
from jax import lax
from jax.experimental import pallas as pl
from jax.experimental.pallas import tpu as pltpu
from jax.experimental.pallas import tpu_sc as plsc

NC = 2
NS = 16
NW = NC * NS
BATCH = 128
IG = 16
KS = 4


def _cnt_kernel(n_pad, d, b_tile):
    mesh = plsc.VectorSubcoreMesh(core_axis_name="c", subcore_axis_name="s")
    rpt = n_pad // NS

    @functools.partial(
        pl.kernel, mesh=mesh,
        out_type=[jax.ShapeDtypeStruct((NC, n_pad, d), jnp.float32)],
        scratch_types=[
            pltpu.VMEM((b_tile, BATCH), jnp.int32),
            pltpu.VMEM((BATCH, d), jnp.float32),
            pltpu.VMEM_SHARED((n_pad, d), jnp.float32),
        ] + [pltpu.SemaphoreType.DMA for _ in range(KS)])
    def cntk(dsts, ones_hbm, zrow, cnt_o, dst_a, ones_v, cnt_sh, *sems):
        c = lax.axis_index("c")
        s = lax.axis_index("s")
        wid = s * NC + c
        pltpu.sync_copy(dsts.at[wid], dst_a)
        pltpu.sync_copy(ones_hbm, ones_v)
        pltpu.sync_copy(zrow, cnt_sh.at[pl.ds(s * rpt, rpt)])
        plsc.subcore_barrier()

        def body(q, carry):
            for k in range(KS):
                j = q * KS + k

                @pl.when(q > 0)
                def _():
                    pltpu.make_async_copy(
                        ones_v, cnt_sh.at[dst_a.at[j]], sems[k]).wait()

                pltpu.async_copy(ones_v, cnt_sh.at[dst_a.at[j]],
                                 sems[k], add=True)
            return carry

        lax.fori_loop(0, b_tile // KS, body, 0)
        for k in range(KS):
            pltpu.make_async_copy(
                ones_v, cnt_sh.at[dst_a.at[b_tile - KS + k]],
                sems[k]).wait()
        plsc.subcore_barrier()
        pltpu.sync_copy(cnt_sh.at[pl.ds(s * rpt, rpt)],
                        cnt_o.at[c, pl.ds(s * rpt, rpt)])
    return cntk


def _seg_sum_kernel(n_pad, d, b_tile, with_cnt=False):
    mesh = plsc.VectorSubcoreMesh(core_axis_name="c", subcore_axis_name="s")
    rpt = n_pad // NS

    assert b_tile % IG == 0
    nq = b_tile // IG
    out_type = [jax.ShapeDtypeStruct((NC, n_pad, d), jnp.float32)]
    scratch = [
        pltpu.VMEM((2 * IG, BATCH), jnp.int32),
        pltpu.VMEM((BATCH, d), jnp.float32),
        pltpu.VMEM((BATCH, d), jnp.float32),
        pltpu.VMEM_SHARED((n_pad, d), jnp.float32),
        pltpu.SemaphoreType.DMA,
        pltpu.SemaphoreType.DMA,
    ]

    @functools.partial(pl.kernel, mesh=mesh, out_type=out_type,
                       scratch_types=scratch)
    def seg(tab, sd, zrow, part_o, sd_v, rows_a, rows_b, acc_sh,
            sem_a, sem_b):
        c = lax.axis_index("c")
        s = lax.axis_index("s")
        wid = s * NC + c
        rows = (rows_a, rows_b)
        sems = (sem_a, sem_b)

        pltpu.sync_copy(zrow, acc_sh.at[pl.ds(s * rpt, rpt)])
        plsc.subcore_barrier()

        def body(g, carry):
            pltpu.sync_copy(sd.at[wid, g], sd_v)
            pltpu.async_copy(tab.at[sd_v.at[0]], rows[0], sems[0])
            for b in range(IG):
                p = b % 2
                if b + 1 < IG:
                    pltpu.async_copy(tab.at[sd_v.at[b + 1]],
                                     rows[1 - p], sems[1 - p])
                pltpu.make_async_copy(tab.at[sd_v.at[b]],
                                      rows[p], sems[p]).wait()
                pltpu.sync_copy(rows[p], acc_sh.at[sd_v.at[IG + b]],
                                add=True)
            return carry

        lax.fori_loop(0, nq, body, 0)
        plsc.subcore_barrier()
        pltpu.sync_copy(acc_sh.at[pl.ds(s * rpt, rpt)],
                        part_o.at[c, pl.ds(s * rpt, rpt)])
    return seg


def _prep_body(x_ref, w_ref, b_ref, a_ref, r_ref):
    y = jnp.dot(x_ref[...], w_ref[...],
                preferred_element_type=jnp.float32,
                precision=lax.Precision.HIGHEST)
    d = a_ref.shape[-1]
    a_ref[...] = y[:, :d]
    r_ref[...] = y[:, d:] + b_ref[...]


def _mid_body(p_ref, pc_ref, r_ref, w_ref, b_ref, a_ref, r2_ref):
    cnt = pc_ref[0, :, 0:1] + pc_ref[1, :, 0:1]
    inv = 1.0 / jnp.maximum(cnt, 1.0)
    h = jnp.maximum((p_ref[0] + p_ref[1]) * inv + r_ref[...], 0.0)
    y = jnp.dot(h, w_ref[...],
                preferred_element_type=jnp.float32,
                precision=lax.Precision.HIGHEST)
    d = a_ref.shape[-1]
    a_ref[...] = y[:, :d]
    r2_ref[...] = y[:, d:] + b_ref[...]


def _post_body(p_ref, pc_ref, r_ref, o_ref):
    cnt = pc_ref[0, :, 0:1] + pc_ref[1, :, 0:1]
    inv = 1.0 / jnp.maximum(cnt, 1.0)
    o_ref[...] = (p_ref[0] + p_ref[1]) * inv + r_ref[...]


def kernel(x, edge_index, edge_weight, Wl1, bl1, Wr1,
           Wl2, bl2, Wr2, Wl3, bl3, Wr3):
    n, d_in = x.shape
    e = edge_index.shape[1]
    d_hid = Wl1.shape[0]
    d_out = Wl3.shape[0]
    n_pad = -(-(n + 1) // 128) * 128
    ndump = n_pad - n

    src = edge_index[0].astype(jnp.int32)
    dst = edge_index[1].astype(jnp.int32)

    unit = NW * BATCH
    b_tile = -(-e // unit)
    b_tile = -(-b_tile // 8) * 8
    e_pad = NW * b_tile * BATCH
    pad = e_pad - e
    pad_ar = jnp.arange(pad, dtype=jnp.int32)
    src_p = jnp.concatenate([src, pad_ar % n])
    dst_p = jnp.concatenate([dst, n + (pad_ar % ndump)])
    srcs = src_p.reshape(NW, b_tile, BATCH)
    dsts = dst_p.reshape(NW, b_tile, BATCH)
    nq = b_tile // IG
    sd = jnp.concatenate([srcs.reshape(NW, nq, IG, BATCH),
                          dsts.reshape(NW, nq, IG, BATCH)], axis=2)

    rpt = n_pad // NS
    zrow = jnp.zeros((rpt, d_hid), jnp.float32)

    cntk = _cnt_kernel(n_pad, d_hid, b_tile)
    seg = _seg_sum_kernel(n_pad, d_hid, b_tile)
    ones128 = jnp.ones((BATCH, d_hid), jnp.float32)

    bn = 1000
    grid = (n // bn,)

    def prep_call(h, Wl, Wr, bl, dh):
        wcat = jnp.concatenate([Wl.T, Wr.T], axis=1)
        return pl.pallas_call(
            _prep_body,
            grid=grid,
            in_specs=[
                pl.BlockSpec((bn, h.shape[1]), lambda i: (i, 0)),
                pl.BlockSpec((h.shape[1], 2 * dh), lambda i: (0, 0)),
                pl.BlockSpec((1, dh), lambda i: (0, 0)),
            ],
            out_specs=[
                pl.BlockSpec((bn, dh), lambda i: (i, 0)),
                pl.BlockSpec((bn, dh), lambda i: (i, 0)),
            ],
            out_shape=[jax.ShapeDtypeStruct((n, dh), jnp.float32)] * 2,
        )(h, wcat, bl.reshape(1, dh))

    def mid_call(part, cntp, r, Wl, Wr, bl, dh):
        wcat = jnp.concatenate([Wl.T, Wr.T], axis=1)
        d_prev = part.shape[2]
        return pl.pallas_call(
            _mid_body,
            grid=grid,
            in_specs=[
                pl.BlockSpec((NC, bn, d_prev), lambda i: (0, i, 0)),
                pl.BlockSpec((NC, bn, 128), lambda i: (0, i, 0)),
                pl.BlockSpec((bn, d_prev), lambda i: (i, 0)),
                pl.BlockSpec((d_prev, 2 * dh), lambda i: (0, 0)),
                pl.BlockSpec((1, dh), lambda i: (0, 0)),
            ],
            out_specs=[
                pl.BlockSpec((bn, dh), lambda i: (i, 0)),
                pl.BlockSpec((bn, dh), lambda i: (i, 0)),
            ],
            out_shape=[jax.ShapeDtypeStruct((n, dh), jnp.float32)] * 2,
        )(part, cntp, r, wcat, bl.reshape(1, dh))

    def post_call(part, cntp, r):
        d_prev = part.shape[2]
        return pl.pallas_call(
            _post_body,
            grid=grid,
            in_specs=[
                pl.BlockSpec((NC, bn, d_prev), lambda i: (0, i, 0)),
                pl.BlockSpec((NC, bn, 128), lambda i: (0, i, 0)),
                pl.BlockSpec((bn, d_prev), lambda i: (i, 0)),
            ],
            out_specs=pl.BlockSpec((bn, d_prev), lambda i: (i, 0)),
            out_shape=jax.ShapeDtypeStruct((n, d_prev), jnp.float32),
        )(part, cntp, r)

    def seg1(*args):
        res = seg(*args)
        return res[0] if isinstance(res, (list, tuple)) else res

    cntp = cntk(dsts, ones128, zrow)
    cntp = cntp[0] if isinstance(cntp, (list, tuple)) else cntp
    a1, r1 = prep_call(x, Wl1, Wr1, bl1, d_hid)
    part1 = seg1(a1, sd, zrow)

    a2, r2 = mid_call(part1, cntp, r1, Wl2, Wr2, bl2, d_hid)
    part2 = seg1(a2, sd, zrow)
    a3, r3 = mid_call(part2, cntp, r2, Wl3, Wr3, bl3, d_out)
    part3 = seg1(a3, sd, zrow)
    out = post_call(part3, cntp, r3)
    return out


# speedup vs baseline: 1.3150x; 1.0375x over previous
import functools

import jax
import jax.numpy as jnp
from jax import lax
from jax.experimental import pallas as pl
from jax.experimental.pallas import tpu as pltpu
from jax.experimental.pallas import tpu_sc as plsc

NC = 2
NS = 16
NW = NC * NS
BATCH = 128
IG = 40
KS = 4


def _cnt_kernel(n_pad, d, b_tile):
    mesh = plsc.VectorSubcoreMesh(core_axis_name="c", subcore_axis_name="s")
    rpt = n_pad // NS

    @functools.partial(
        pl.kernel, mesh=mesh,
        out_type=[jax.ShapeDtypeStruct((NC, n_pad, d), jnp.float32)],
        scratch_types=[
            pltpu.VMEM((b_tile, BATCH), jnp.int32),
            pltpu.VMEM((BATCH, d), jnp.float32),
            pltpu.VMEM_SHARED((n_pad, d), jnp.float32),
        ] + [pltpu.SemaphoreType.DMA for _ in range(KS)])
    def cntk(dsts, ones_hbm, zrow, cnt_o, dst_a, ones_v, cnt_sh, *sems):
        c = lax.axis_index("c")
        s = lax.axis_index("s")
        wid = s * NC + c
        pltpu.sync_copy(dsts.at[wid], dst_a)
        pltpu.sync_copy(ones_hbm, ones_v)
        pltpu.sync_copy(zrow, cnt_sh.at[pl.ds(s * rpt, rpt)])
        plsc.subcore_barrier()

        def body(q, carry):
            for k in range(KS):
                j = q * KS + k

                @pl.when(q > 0)
                def _():
                    pltpu.make_async_copy(
                        ones_v, cnt_sh.at[dst_a.at[j]], sems[k]).wait()

                pltpu.async_copy(ones_v, cnt_sh.at[dst_a.at[j]],
                                 sems[k], add=True)
            return carry

        lax.fori_loop(0, b_tile // KS, body, 0)
        for k in range(KS):
            pltpu.make_async_copy(
                ones_v, cnt_sh.at[dst_a.at[b_tile - KS + k]],
                sems[k]).wait()
        plsc.subcore_barrier()
        pltpu.sync_copy(cnt_sh.at[pl.ds(s * rpt, rpt)],
                        cnt_o.at[c, pl.ds(s * rpt, rpt)])
    return cntk


def _seg_sum_kernel(n_pad, d, b_tile, with_cnt=False):
    mesh = plsc.VectorSubcoreMesh(core_axis_name="c", subcore_axis_name="s")
    rpt = n_pad // NS

    assert b_tile % IG == 0
    nq = b_tile // IG
    out_type = [jax.ShapeDtypeStruct((NC, n_pad, d), jnp.float32)]
    scratch = [
        pltpu.VMEM((2 * IG, BATCH), jnp.int32),
        pltpu.VMEM((BATCH, d), jnp.float32),
        pltpu.VMEM((BATCH, d), jnp.float32),
        pltpu.VMEM_SHARED((n_pad, d), jnp.float32),
        pltpu.SemaphoreType.DMA,
        pltpu.SemaphoreType.DMA,
    ]

    @functools.partial(pl.kernel, mesh=mesh, out_type=out_type,
                       scratch_types=scratch)
    def seg(tab, sd, zrow, part_o, sd_v, rows_a, rows_b, acc_sh,
            sem_a, sem_b):
        c = lax.axis_index("c")
        s = lax.axis_index("s")
        wid = s * NC + c
        rows = (rows_a, rows_b)
        sems = (sem_a, sem_b)

        pltpu.sync_copy(zrow, acc_sh.at[pl.ds(s * rpt, rpt)])
        plsc.subcore_barrier()

        def body(g, carry):
            pltpu.sync_copy(sd.at[wid, g], sd_v)
            pltpu.async_copy(tab.at[sd_v.at[0]], rows[0], sems[0])
            for b in range(IG):
                p = b % 2
                if b + 1 < IG:
                    pltpu.async_copy(tab.at[sd_v.at[b + 1]],
                                     rows[1 - p], sems[1 - p])
                pltpu.make_async_copy(tab.at[sd_v.at[b]],
                                      rows[p], sems[p]).wait()
                pltpu.sync_copy(rows[p], acc_sh.at[sd_v.at[IG + b]],
                                add=True)
            return carry

        lax.fori_loop(0, nq, body, 0)
        plsc.subcore_barrier()
        pltpu.sync_copy(acc_sh.at[pl.ds(s * rpt, rpt)],
                        part_o.at[c, pl.ds(s * rpt, rpt)])
    return seg


def _prep_body(x_ref, w_ref, b_ref, a_ref, r_ref):
    y = jnp.dot(x_ref[...], w_ref[...],
                preferred_element_type=jnp.float32,
                precision=lax.Precision.HIGHEST)
    d = a_ref.shape[-1]
    a_ref[...] = y[:, :d]
    r_ref[...] = y[:, d:] + b_ref[...]


def _mid_body(p_ref, pc_ref, r_ref, w_ref, b_ref, a_ref, r2_ref):
    cnt = pc_ref[0, :, 0:1] + pc_ref[1, :, 0:1]
    inv = 1.0 / jnp.maximum(cnt, 1.0)
    h = jnp.maximum((p_ref[0] + p_ref[1]) * inv + r_ref[...], 0.0)
    y = jnp.dot(h, w_ref[...],
                preferred_element_type=jnp.float32,
                precision=lax.Precision.HIGHEST)
    d = a_ref.shape[-1]
    a_ref[...] = y[:, :d]
    r2_ref[...] = y[:, d:] + b_ref[...]


def _post_body(p_ref, pc_ref, r_ref, o_ref):
    cnt = pc_ref[0, :, 0:1] + pc_ref[1, :, 0:1]
    inv = 1.0 / jnp.maximum(cnt, 1.0)
    o_ref[...] = (p_ref[0] + p_ref[1]) * inv + r_ref[...]


def kernel(x, edge_index, edge_weight, Wl1, bl1, Wr1,
           Wl2, bl2, Wr2, Wl3, bl3, Wr3):
    n, d_in = x.shape
    e = edge_index.shape[1]
    d_hid = Wl1.shape[0]
    d_out = Wl3.shape[0]
    n_pad = -(-(n + 1) // 128) * 128
    ndump = n_pad - n

    src = edge_index[0].astype(jnp.int32)
    dst = edge_index[1].astype(jnp.int32)

    unit = NW * BATCH
    b_tile = -(-e // unit)
    b_tile = -(-b_tile // 8) * 8
    e_pad = NW * b_tile * BATCH
    pad = e_pad - e
    pad_ar = jnp.arange(pad, dtype=jnp.int32)
    src_p = jnp.concatenate([src, pad_ar % n])
    dst_p = jnp.concatenate([dst, n + (pad_ar % ndump)])
    srcs = src_p.reshape(NW, b_tile, BATCH)
    dsts = dst_p.reshape(NW, b_tile, BATCH)
    nq = b_tile // IG
    sd = jnp.concatenate([srcs.reshape(NW, nq, IG, BATCH),
                          dsts.reshape(NW, nq, IG, BATCH)], axis=2)

    rpt = n_pad // NS
    zrow = jnp.zeros((rpt, d_hid), jnp.float32)

    cntk = _cnt_kernel(n_pad, d_hid, b_tile)
    seg = _seg_sum_kernel(n_pad, d_hid, b_tile)
    ones128 = jnp.ones((BATCH, d_hid), jnp.float32)

    bn = 1000
    grid = (n // bn,)

    def prep_call(h, Wl, Wr, bl, dh):
        wcat = jnp.concatenate([Wl.T, Wr.T], axis=1)
        return pl.pallas_call(
            _prep_body,
            grid=grid,
            in_specs=[
                pl.BlockSpec((bn, h.shape[1]), lambda i: (i, 0)),
                pl.BlockSpec((h.shape[1], 2 * dh), lambda i: (0, 0)),
                pl.BlockSpec((1, dh), lambda i: (0, 0)),
            ],
            out_specs=[
                pl.BlockSpec((bn, dh), lambda i: (i, 0)),
                pl.BlockSpec((bn, dh), lambda i: (i, 0)),
            ],
            out_shape=[jax.ShapeDtypeStruct((n, dh), jnp.float32)] * 2,
        )(h, wcat, bl.reshape(1, dh))

    def mid_call(part, cntp, r, Wl, Wr, bl, dh):
        wcat = jnp.concatenate([Wl.T, Wr.T], axis=1)
        d_prev = part.shape[2]
        return pl.pallas_call(
            _mid_body,
            grid=grid,
            in_specs=[
                pl.BlockSpec((NC, bn, d_prev), lambda i: (0, i, 0)),
                pl.BlockSpec((NC, bn, 128), lambda i: (0, i, 0)),
                pl.BlockSpec((bn, d_prev), lambda i: (i, 0)),
                pl.BlockSpec((d_prev, 2 * dh), lambda i: (0, 0)),
                pl.BlockSpec((1, dh), lambda i: (0, 0)),
            ],
            out_specs=[
                pl.BlockSpec((bn, dh), lambda i: (i, 0)),
                pl.BlockSpec((bn, dh), lambda i: (i, 0)),
            ],
            out_shape=[jax.ShapeDtypeStruct((n, dh), jnp.float32)] * 2,
        )(part, cntp, r, wcat, bl.reshape(1, dh))

    def post_call(part, cntp, r):
        d_prev = part.shape[2]
        return pl.pallas_call(
            _post_body,
            grid=grid,
            in_specs=[
                pl.BlockSpec((NC, bn, d_prev), lambda i: (0, i, 0)),
                pl.BlockSpec((NC, bn, 128), lambda i: (0, i, 0)),
                pl.BlockSpec((bn, d_prev), lambda i: (i, 0)),
            ],
            out_specs=pl.BlockSpec((bn, d_prev), lambda i: (i, 0)),
            out_shape=jax.ShapeDtypeStruct((n, d_prev), jnp.float32),
        )(part, cntp, r)

    def seg1(*args):
        res = seg(*args)
        return res[0] if isinstance(res, (list, tuple)) else res

    cntp = cntk(dsts, ones128, zrow)
    cntp = cntp[0] if isinstance(cntp, (list, tuple)) else cntp
    a1, r1 = prep_call(x, Wl1, Wr1, bl1, d_hid)
    part1 = seg1(a1, sd, zrow)

    a2, r2 = mid_call(part1, cntp, r1, Wl2, Wr2, bl2, d_hid)
    part2 = seg1(a2, sd, zrow)
    a3, r3 = mid_call(part2, cntp, r2, Wl3, Wr3, bl3, d_out)
    part3 = seg1(a3, sd, zrow)
    out = post_call(part3, cntp, r3)
    return out
